# baseline (speedup 1.0000x reference)
import functools

import jax
import jax.numpy as jnp
from jax import lax
from jax.experimental import pallas as pl
from jax.experimental.pallas import tpu as pltpu
from jax.experimental.pallas import tpu_sc as plsc

N = 10000
E = 320000
D_IN = 128
D_HID = 256
D_OUT = 128

NC = 2
NS = 16
ACC_ROWS = 10240
ROWS_PER_TILE = ACC_ROWS // NS
CH = 80
CHD = 80
ZROWS = 40


def _mesh():
    return plsc.VectorSubcoreMesh(core_axis_name="c", subcore_axis_name="s")


@functools.partial(
    pl.kernel,
    mesh=_mesh(),
    out_type=jax.ShapeDtypeStruct((NC * ACC_ROWS,), jnp.float32),
    scratch_types=[
        pltpu.VMEM((CHD,), jnp.int32),
        pltpu.VMEM((CHD,), jnp.int32),
        pltpu.VMEM((CHD,), jnp.float32),
        pltpu.VMEM((ROWS_PER_TILE,), jnp.float32),
        pltpu.VMEM_SHARED((ACC_ROWS,), jnp.float32),
        pltpu.SemaphoreType.DMA,
        pltpu.SemaphoreType.DMA,
    ],
)
def _deg_sc(dst_hbm, out_hbm, dv0, dv1, ones, zflat, dacc, si0, si1):
    c = lax.axis_index("c")
    s = lax.axis_index("s")
    one16 = jnp.ones((16,), jnp.float32)
    zero16 = jnp.zeros((16,), jnp.float32)
    for j in range(CHD // 16):
        ones[pl.ds(j * 16, 16)] = one16

    def zinit(i, _):
        zflat[pl.ds(i * 16, 16)] = zero16
        return 0

    lax.fori_loop(0, ROWS_PER_TILE // 16, zinit, 0)
    pltpu.sync_copy(zflat, dacc.at[pl.ds(s * ROWS_PER_TILE, ROWS_PER_TILE)])
    plsc.subcore_barrier()

    epc = E // NC
    ept = epc // NS
    nch = ept // CHD
    first = nch % 2
    ebase = c * epc + s * ept

    def istart(g, dv, sem):
        b = ebase + jnp.minimum(g, nch - 1) * CHD
        pltpu.async_copy(dst_hbm.at[pl.ds(b, CHD)], dv, sem)

    def iwait(dv, sem):
        pltpu.make_async_copy(dst_hbm.at[pl.ds(0, CHD)], dv, sem).wait()

    if first:
        pltpu.sync_copy(dst_hbm.at[pl.ds(ebase, CHD)], dv0)
        pltpu.sync_copy(ones, dacc.at[dv0], add=True)
    istart(first, dv0, si0)
    istart(first + 1, dv1, si1)

    def pair(k, _):
        a = first + 2 * k
        iwait(dv0, si0)
        pltpu.sync_copy(ones, dacc.at[dv0], add=True)
        istart(a + 2, dv0, si0)
        iwait(dv1, si1)
        pltpu.sync_copy(ones, dacc.at[dv1], add=True)
        istart(a + 3, dv1, si1)
        return 0

    lax.fori_loop(0, (nch - first) // 2, pair, 0)
    iwait(dv0, si0)
    iwait(dv1, si1)
    plsc.subcore_barrier()
    pltpu.sync_copy(
        dacc.at[pl.ds(s * ROWS_PER_TILE, ROWS_PER_TILE)],
        out_hbm.at[pl.ds(c * ACC_ROWS + s * ROWS_PER_TILE, ROWS_PER_TILE)],
    )


def _make_agg(n_tbl_rows, d, edges_per_tile, core_edge_stride, core_tbl_stride,
              ch):
    n_chunks = edges_per_tile // ch
    tail = edges_per_tile - n_chunks * ch
    assert ch <= 128 and ch % 16 == 0
    assert tail % 16 == 0

    @functools.partial(
        pl.kernel,
        mesh=_mesh(),
        out_type=jax.ShapeDtypeStruct((NC * ACC_ROWS, d), jnp.float32),
        scratch_types=[
            pltpu.VMEM((ch,), jnp.int32),
            pltpu.VMEM((ch,), jnp.int32),
            pltpu.VMEM((ch,), jnp.int32),
            pltpu.VMEM((ch,), jnp.int32),
            pltpu.VMEM((ch,), jnp.int32),
            pltpu.VMEM((ch,), jnp.int32),
            pltpu.VMEM((max(tail, 16),), jnp.int32),
            pltpu.VMEM((max(tail, 16),), jnp.int32),
            pltpu.VMEM((max(tail, 16),), jnp.int32),
            pltpu.VMEM((max(tail, 16), d), jnp.float32),
            pltpu.VMEM((ch,), jnp.int32),
            pltpu.VMEM((ch,), jnp.int32),
            pltpu.VMEM((ch, d), jnp.float32),
            pltpu.VMEM((ch, d), jnp.float32),
            pltpu.VMEM((ZROWS, d), jnp.float32),
            pltpu.VMEM_SHARED((ACC_ROWS, d), jnp.float32),
            pltpu.SemaphoreType.DMA,
            pltpu.SemaphoreType.DMA,
            pltpu.SemaphoreType.DMA,
            pltpu.SemaphoreType.DMA,
            pltpu.SemaphoreType.DMA,
            pltpu.SemaphoreType.DMA,
        ],
    )
    def agg(src_hbm, dst_hbm, tbl_hbm, out_hbm, sv0, sv1, gv0, gv1, dv0, dv1,
            svt, gvt, dvt, rowst, dS0, dS1, rows0, rows1, zbuf, acc,
            si0, si1, sg0, sg1, ss0, ss1):
        c = lax.axis_index("c")
        s = lax.axis_index("s")
        zero16 = jnp.zeros((16,), jnp.float32)
        vecs_per_row = d // 16

        def zinit(i, _):
            zbuf[i // vecs_per_row, pl.ds((i % vecs_per_row) * 16, 16)] = zero16
            return 0

        lax.fori_loop(0, ZROWS * vecs_per_row, zinit, 0)

        def zcopy(j, _):
            pltpu.sync_copy(
                zbuf, acc.at[pl.ds(s * ROWS_PER_TILE + j * ZROWS, ZROWS)])
            return 0

        lax.fori_loop(0, ROWS_PER_TILE // ZROWS, zcopy, 0)
        plsc.subcore_barrier()

        ebase = c * core_edge_stride + s * edges_per_tile
        toff = jnp.broadcast_to((c * core_tbl_stride).astype(jnp.int32), (16,))

        def idx_start(g, sv, dv, sem):
            b = ebase + jnp.minimum(g, n_chunks - 1) * ch
            pltpu.async_copy(src_hbm.at[pl.ds(b, ch)], sv, sem)
            pltpu.async_copy(dst_hbm.at[pl.ds(b, ch)], dv, sem)

        def idx_wait(sv, dv, sem):
            pltpu.make_async_copy(src_hbm.at[pl.ds(0, ch)], sv, sem).wait()
            pltpu.make_async_copy(dst_hbm.at[pl.ds(0, ch)], dv, sem).wait()

        def adjust(sv, gv):
            if core_tbl_stride:
                for j in range(ch // 16):
                    gv[pl.ds(j * 16, 16)] = sv[pl.ds(j * 16, 16)] + toff

        def _idxref(sv, gv):
            return gv if core_tbl_stride else sv

        def gather_start(sv, gv, rows, sem):
            pltpu.async_copy(tbl_hbm.at[_idxref(sv, gv)], rows, sem)

        def gather_wait(sv, gv, rows, sem):
            pltpu.make_async_copy(
                tbl_hbm.at[_idxref(sv, gv)], rows, sem).wait()

        def copy_didx(dv, dS):
            for j in range(ch // 16):
                dS[pl.ds(j * 16, 16)] = dv[pl.ds(j * 16, 16)]

        def scatter_start(dS, rows, sem):
            pltpu.async_copy(rows, acc.at[dS], sem, add=True)

        def scatter_wait(dS, rows, sem):
            pltpu.make_async_copy(rows, acc.at[dS], sem).wait()

        if tail:
            tb = ebase + n_chunks * ch
            pltpu.sync_copy(src_hbm.at[pl.ds(tb, tail)], svt)
            pltpu.sync_copy(dst_hbm.at[pl.ds(tb, tail)], dvt)
            if core_tbl_stride:
                for j in range(tail // 16):
                    gvt[pl.ds(j * 16, 16)] = svt[pl.ds(j * 16, 16)] + toff
            tref = gvt if core_tbl_stride else svt
            pltpu.async_copy(tbl_hbm.at[tref], rowst, sg0).wait()
            pltpu.sync_copy(rowst, acc.at[dvt], add=True)

        serial_n = 1 if n_chunks % 2 == 0 else 0
        for t in range(serial_n):
            pltpu.sync_copy(src_hbm.at[pl.ds(ebase + t * ch, ch)], sv0)
            pltpu.sync_copy(dst_hbm.at[pl.ds(ebase + t * ch, ch)], dv0)
            adjust(sv0, gv0)
            gather_start(sv0, gv0, rows0, sg0)
            gather_wait(sv0, gv0, rows0, sg0)
            pltpu.sync_copy(rows0, acc.at[dv0], add=True)

        q = serial_n
        pltpu.sync_copy(src_hbm.at[pl.ds(ebase + q * ch, ch)], sv1)
        pltpu.sync_copy(dst_hbm.at[pl.ds(ebase + q * ch, ch)], dv1)
        adjust(sv1, gv1)
        copy_didx(dv1, dS1)
        gather_start(sv1, gv1, rows1, sg1)
        gather_wait(sv1, gv1, rows1, sg1)
        scatter_start(dS1, rows1, ss1)
        pltpu.sync_copy(src_hbm.at[pl.ds(ebase + (q + 1) * ch, ch)], sv0)
        pltpu.sync_copy(dst_hbm.at[pl.ds(ebase + (q + 1) * ch, ch)], dv0)
        adjust(sv0, gv0)
        copy_didx(dv0, dS0)
        gather_start(sv0, gv0, rows0, sg0)
        idx_start(q + 2, sv1, dv1, si1)

        def pair(k, _):
            a = q + 1 + 2 * k
            idx_wait(sv1, dv1, si1)
            adjust(sv1, gv1)
            gather_wait(sv0, gv0, rows0, sg0)
            scatter_wait(dS1, rows1, ss1)
            copy_didx(dv1, dS1)
            gather_start(sv1, gv1, rows1, sg1)
            scatter_start(dS0, rows0, ss0)
            idx_start(a + 2, sv0, dv0, si0)
            gather_wait(sv1, gv1, rows1, sg1)
            idx_wait(sv0, dv0, si0)
            adjust(sv0, gv0)
            scatter_wait(dS0, rows0, ss0)
            copy_didx(dv0, dS0)
            gather_start(sv0, gv0, rows0, sg0)
            scatter_start(dS1, rows1, ss1)
            idx_start(a + 3, sv1, dv1, si1)
            return 0

        lax.fori_loop(0, (n_chunks - q - 1) // 2, pair, 0)
        gather_wait(sv0, gv0, rows0, sg0)
        scatter_wait(dS1, rows1, ss1)
        idx_wait(sv1, dv1, si1)
        plsc.subcore_barrier()
        pltpu.sync_copy(
            acc.at[pl.ds(s * ROWS_PER_TILE, ROWS_PER_TILE)],
            out_hbm.at[pl.ds(c * ACC_ROWS + s * ROWS_PER_TILE, ROWS_PER_TILE)],
        )

    return agg


_agg1 = _make_agg(2 * N, D_HID // 2, E // NS, 0, N, 128)
_agg2 = _make_agg(N, D_OUT, E // (NC * NS), E // NC, 0, 128)


_RB = 1000


def _dis_from(deg_ref):
    deg = deg_ref[:, 0] + deg_ref[:, 1]
    return jnp.where(deg > 0, lax.rsqrt(jnp.maximum(deg, 1e-12)), 0.0)


def _tc0_body(x_ref, w1_ref, out_ref):
    out_ref[...] = jnp.dot(
        x_ref[...], w1_ref[...], preferred_element_type=jnp.float32)


_tc0 = pl.pallas_call(
    _tc0_body,
    grid=(N // _RB,),
    in_specs=[
        pl.BlockSpec((_RB, D_IN), lambda i: (i, 0)),
        pl.BlockSpec((D_IN, D_HID), lambda i: (0, 0)),
    ],
    out_specs=pl.BlockSpec((_RB, D_HID), lambda i: (i, 0)),
    out_shape=jax.ShapeDtypeStruct((N, D_HID), jnp.float32),
)


def _tc1_body(h_ref, deg_ref, out_ref):
    dis = _dis_from(deg_ref)
    hs = h_ref[...] * dis[:, None]
    out_ref[0] = hs[:, : D_HID // 2]
    out_ref[1] = hs[:, D_HID // 2:]


_tc1 = pl.pallas_call(
    _tc1_body,
    grid=(N // _RB,),
    in_specs=[
        pl.BlockSpec((_RB, D_HID), lambda i: (i, 0)),
        pl.BlockSpec((_RB, 2), lambda i: (i, 0)),
    ],
    out_specs=pl.BlockSpec((2, _RB, D_HID // 2), lambda i: (0, i, 0)),
    out_shape=jax.ShapeDtypeStruct((2, N, D_HID // 2), jnp.float32),
)


def _tc2_body(raw_ref, deg_ref, b1_ref, w2_ref, out_ref):
    dis = _dis_from(deg_ref)
    raw = raw_ref[...]
    h = jnp.concatenate([raw[0], raw[1]], axis=1) * dis[:, None] + b1_ref[...]
    h = jnp.maximum(h, 0.0)
    h2 = jnp.dot(h, w2_ref[...], preferred_element_type=jnp.float32)
    out_ref[...] = h2 * dis[:, None]


_tc2 = pl.pallas_call(
    _tc2_body,
    grid=(N // _RB,),
    in_specs=[
        pl.BlockSpec((2, _RB, D_HID // 2), lambda i: (0, i, 0)),
        pl.BlockSpec((_RB, 2), lambda i: (i, 0)),
        pl.BlockSpec((1, D_HID), lambda i: (0, 0)),
        pl.BlockSpec((D_HID, D_OUT), lambda i: (0, 0)),
    ],
    out_specs=pl.BlockSpec((_RB, D_OUT), lambda i: (i, 0)),
    out_shape=jax.ShapeDtypeStruct((N, D_OUT), jnp.float32),
)


def _tc3_body(raw_ref, deg_ref, b2_ref, out_ref):
    dis = _dis_from(deg_ref)
    raw = raw_ref[...]
    out_ref[...] = (raw[0] + raw[1]) * dis[:, None] + b2_ref[...]


_tc3 = pl.pallas_call(
    _tc3_body,
    grid=(N // _RB,),
    in_specs=[
        pl.BlockSpec((2, _RB, D_OUT), lambda i: (0, i, 0)),
        pl.BlockSpec((_RB, 2), lambda i: (i, 0)),
        pl.BlockSpec((1, D_OUT), lambda i: (0, 0)),
    ],
    out_specs=pl.BlockSpec((_RB, D_OUT), lambda i: (i, 0)),
    out_shape=jax.ShapeDtypeStruct((N, D_OUT), jnp.float32),
)


def kernel(x, edge_index, W1, b1, W2, b2):
    ei = edge_index.astype(jnp.int32)
    src = ei[0]
    dst = ei[1]

    h1 = _tc0(x, W1)
    deg_pad = _deg_sc(dst)
    deg2 = deg_pad.reshape(NC, ACC_ROWS)[:, :N].T
    h1s_split = _tc1(h1, deg2)
    table1 = h1s_split.reshape(2 * N, D_HID // 2)
    raw1_pad = _agg1(src, dst, table1)
    raw1 = raw1_pad.reshape(NC, ACC_ROWS, D_HID // 2)[:, :N]

    h2s = _tc2(raw1, deg2, b1.reshape(1, D_HID), W2)
    raw2_pad = _agg2(src, dst, h2s)
    raw2 = raw2_pad.reshape(NC, ACC_ROWS, D_OUT)[:, :N]

    return _tc3(raw2, deg2, b2.reshape(1, D_OUT))


# speedup vs baseline: 1.0106x; 1.0106x over previous
import functools

import jax
import jax.numpy as jnp
from jax import lax
from jax.experimental import pallas as pl
from jax.experimental.pallas import tpu as pltpu
from jax.experimental.pallas import tpu_sc as plsc

N = 10000
E = 320000
D_IN = 128
D_HID = 256
D_OUT = 128

NC = 2
NS = 16
ACC_ROWS = 10240
ROWS_PER_TILE = ACC_ROWS // NS
CH = 80
CHD = 80
ZROWS = 40


def _mesh():
    return plsc.VectorSubcoreMesh(core_axis_name="c", subcore_axis_name="s")


@functools.partial(
    pl.kernel,
    mesh=_mesh(),
    out_type=jax.ShapeDtypeStruct((NC * ACC_ROWS,), jnp.float32),
    scratch_types=[
        pltpu.VMEM((CHD,), jnp.int32),
        pltpu.VMEM((CHD,), jnp.int32),
        pltpu.VMEM((CHD,), jnp.float32),
        pltpu.VMEM((ROWS_PER_TILE,), jnp.float32),
        pltpu.VMEM_SHARED((ACC_ROWS,), jnp.float32),
        pltpu.SemaphoreType.DMA,
        pltpu.SemaphoreType.DMA,
    ],
)
def _deg_sc(dst_hbm, out_hbm, dv0, dv1, ones, zflat, dacc, si0, si1):
    c = lax.axis_index("c")
    s = lax.axis_index("s")
    one16 = jnp.ones((16,), jnp.float32)
    zero16 = jnp.zeros((16,), jnp.float32)
    for j in range(CHD // 16):
        ones[pl.ds(j * 16, 16)] = one16

    def zinit(i, _):
        zflat[pl.ds(i * 16, 16)] = zero16
        return 0

    lax.fori_loop(0, ROWS_PER_TILE // 16, zinit, 0)
    pltpu.sync_copy(zflat, dacc.at[pl.ds(s * ROWS_PER_TILE, ROWS_PER_TILE)])
    plsc.subcore_barrier()

    epc = E // NC
    ept = epc // NS
    nch = ept // CHD
    first = nch % 2
    ebase = c * epc + s * ept

    def istart(g, dv, sem):
        b = ebase + jnp.minimum(g, nch - 1) * CHD
        pltpu.async_copy(dst_hbm.at[pl.ds(b, CHD)], dv, sem)

    def iwait(dv, sem):
        pltpu.make_async_copy(dst_hbm.at[pl.ds(0, CHD)], dv, sem).wait()

    if first:
        pltpu.sync_copy(dst_hbm.at[pl.ds(ebase, CHD)], dv0)
        pltpu.sync_copy(ones, dacc.at[dv0], add=True)
    istart(first, dv0, si0)
    istart(first + 1, dv1, si1)

    def pair(k, _):
        a = first + 2 * k
        iwait(dv0, si0)
        pltpu.sync_copy(ones, dacc.at[dv0], add=True)
        istart(a + 2, dv0, si0)
        iwait(dv1, si1)
        pltpu.sync_copy(ones, dacc.at[dv1], add=True)
        istart(a + 3, dv1, si1)
        return 0

    lax.fori_loop(0, (nch - first) // 2, pair, 0)
    iwait(dv0, si0)
    iwait(dv1, si1)
    plsc.subcore_barrier()
    pltpu.sync_copy(
        dacc.at[pl.ds(s * ROWS_PER_TILE, ROWS_PER_TILE)],
        out_hbm.at[pl.ds(c * ACC_ROWS + s * ROWS_PER_TILE, ROWS_PER_TILE)],
    )


def _make_agg(n_tbl_rows, d, edges_per_tile, core_edge_stride, core_tbl_stride,
              ch):
    n_chunks = edges_per_tile // ch
    tail = edges_per_tile - n_chunks * ch
    assert ch <= 128 and ch % 16 == 0
    assert tail % 16 == 0

    @functools.partial(
        pl.kernel,
        mesh=_mesh(),
        out_type=jax.ShapeDtypeStruct((NC * ACC_ROWS, d), jnp.float32),
        scratch_types=[
            pltpu.VMEM((ch,), jnp.int32),
            pltpu.VMEM((ch,), jnp.int32),
            pltpu.VMEM((ch,), jnp.int32),
            pltpu.VMEM((ch,), jnp.int32),
            pltpu.VMEM((ch,), jnp.int32),
            pltpu.VMEM((ch,), jnp.int32),
            pltpu.VMEM((max(tail, 16),), jnp.int32),
            pltpu.VMEM((max(tail, 16),), jnp.int32),
            pltpu.VMEM((max(tail, 16),), jnp.int32),
            pltpu.VMEM((max(tail, 16), d), jnp.float32),
            pltpu.VMEM((ch, d), jnp.float32),
            pltpu.VMEM((ch, d), jnp.float32),
            pltpu.VMEM((ZROWS, d), jnp.float32),
            pltpu.VMEM_SHARED((ACC_ROWS, d), jnp.float32),
            pltpu.SemaphoreType.DMA,
            pltpu.SemaphoreType.DMA,
            pltpu.SemaphoreType.DMA,
            pltpu.SemaphoreType.DMA,
        ],
    )
    def agg(src_hbm, dst_hbm, tbl_hbm, out_hbm, sv0, sv1, gv0, gv1, dv0, dv1,
            svt, gvt, dvt, rowst, rows0, rows1, zbuf, acc, si0, si1, sg0, sg1):
        c = lax.axis_index("c")
        s = lax.axis_index("s")
        zero16 = jnp.zeros((16,), jnp.float32)
        vecs_per_row = d // 16

        def zinit(i, _):
            zbuf[i // vecs_per_row, pl.ds((i % vecs_per_row) * 16, 16)] = zero16
            return 0

        lax.fori_loop(0, ZROWS * vecs_per_row, zinit, 0)

        def zcopy(j, _):
            pltpu.sync_copy(
                zbuf, acc.at[pl.ds(s * ROWS_PER_TILE + j * ZROWS, ZROWS)])
            return 0

        lax.fori_loop(0, ROWS_PER_TILE // ZROWS, zcopy, 0)
        plsc.subcore_barrier()

        ebase = c * core_edge_stride + s * edges_per_tile
        toff = jnp.broadcast_to((c * core_tbl_stride).astype(jnp.int32), (16,))

        def idx_start(g, sv, dv, sem):
            b = ebase + jnp.minimum(g, n_chunks - 1) * ch
            pltpu.async_copy(src_hbm.at[pl.ds(b, ch)], sv, sem)
            pltpu.async_copy(dst_hbm.at[pl.ds(b, ch)], dv, sem)

        def idx_wait(sv, dv, sem):
            pltpu.make_async_copy(src_hbm.at[pl.ds(0, ch)], sv, sem).wait()
            pltpu.make_async_copy(dst_hbm.at[pl.ds(0, ch)], dv, sem).wait()

        def adjust(sv, gv):
            if core_tbl_stride:
                for j in range(ch // 16):
                    gv[pl.ds(j * 16, 16)] = sv[pl.ds(j * 16, 16)] + toff

        def _idxref(sv, gv):
            return gv if core_tbl_stride else sv

        def gather_start(sv, gv, rows, sem):
            pltpu.async_copy(tbl_hbm.at[_idxref(sv, gv)], rows, sem)

        def gather_wait(sv, gv, rows, sem):
            pltpu.make_async_copy(
                tbl_hbm.at[_idxref(sv, gv)], rows, sem).wait()

        def scatter(dv, rows):
            pltpu.sync_copy(rows, acc.at[dv], add=True)

        if tail:
            tb = ebase + n_chunks * ch
            pltpu.sync_copy(src_hbm.at[pl.ds(tb, tail)], svt)
            pltpu.sync_copy(dst_hbm.at[pl.ds(tb, tail)], dvt)
            if core_tbl_stride:
                for j in range(tail // 16):
                    gvt[pl.ds(j * 16, 16)] = svt[pl.ds(j * 16, 16)] + toff
            tref = gvt if core_tbl_stride else svt
            pltpu.async_copy(tbl_hbm.at[tref], rowst, sg0).wait()
            pltpu.sync_copy(rowst, acc.at[dvt], add=True)

        first = n_chunks % 2
        if first:
            pltpu.sync_copy(src_hbm.at[pl.ds(ebase, ch)], sv0)
            pltpu.sync_copy(dst_hbm.at[pl.ds(ebase, ch)], dv0)
            adjust(sv0, gv0)
            gather_start(sv0, gv0, rows0, sg0)
            gather_wait(sv0, gv0, rows0, sg0)
            scatter(dv0, rows0)
        pltpu.sync_copy(src_hbm.at[pl.ds(ebase + first * ch, ch)], sv0)
        pltpu.sync_copy(dst_hbm.at[pl.ds(ebase + first * ch, ch)], dv0)
        adjust(sv0, gv0)
        gather_start(sv0, gv0, rows0, sg0)
        idx_start(first + 1, sv1, dv1, si1)

        def pair(k, _):
            a = first + 2 * k
            gather_wait(sv0, gv0, rows0, sg0)
            idx_wait(sv1, dv1, si1)
            adjust(sv1, gv1)
            gather_start(sv1, gv1, rows1, sg1)
            scatter(dv0, rows0)
            idx_start(a + 2, sv0, dv0, si0)
            gather_wait(sv1, gv1, rows1, sg1)
            idx_wait(sv0, dv0, si0)
            adjust(sv0, gv0)
            gather_start(sv0, gv0, rows0, sg0)
            scatter(dv1, rows1)
            idx_start(a + 3, sv1, dv1, si1)
            return 0

        lax.fori_loop(0, (n_chunks - first) // 2, pair, 0)
        idx_wait(sv1, dv1, si1)
        gather_wait(sv0, gv0, rows0, sg0)
        plsc.subcore_barrier()
        pltpu.sync_copy(
            acc.at[pl.ds(s * ROWS_PER_TILE, ROWS_PER_TILE)],
            out_hbm.at[pl.ds(c * ACC_ROWS + s * ROWS_PER_TILE, ROWS_PER_TILE)],
        )

    return agg


_agg1 = _make_agg(2 * N, D_HID // 2, E // NS, 0, N, 128)
_agg2 = _make_agg(N, D_OUT, E // (NC * NS), E // NC, 0, 128)


_RB = 1000


def _dis_from(deg_ref):
    deg = deg_ref[:, 0] + deg_ref[:, 1]
    return jnp.where(deg > 0, lax.rsqrt(jnp.maximum(deg, 1e-12)), 0.0)


def _tc0_body(x_ref, w1_ref, out_ref):
    out_ref[...] = jnp.dot(
        x_ref[...], w1_ref[...], preferred_element_type=jnp.float32)


_tc0 = pl.pallas_call(
    _tc0_body,
    grid=(N // _RB,),
    in_specs=[
        pl.BlockSpec((_RB, D_IN), lambda i: (i, 0)),
        pl.BlockSpec((D_IN, D_HID), lambda i: (0, 0)),
    ],
    out_specs=pl.BlockSpec((_RB, D_HID), lambda i: (i, 0)),
    out_shape=jax.ShapeDtypeStruct((N, D_HID), jnp.float32),
)


def _tc1_body(h_ref, deg_ref, out_ref):
    dis = _dis_from(deg_ref)
    hs = h_ref[...] * dis[:, None]
    out_ref[0] = hs[:, : D_HID // 2]
    out_ref[1] = hs[:, D_HID // 2:]


_tc1 = pl.pallas_call(
    _tc1_body,
    grid=(N // _RB,),
    in_specs=[
        pl.BlockSpec((_RB, D_HID), lambda i: (i, 0)),
        pl.BlockSpec((_RB, 2), lambda i: (i, 0)),
    ],
    out_specs=pl.BlockSpec((2, _RB, D_HID // 2), lambda i: (0, i, 0)),
    out_shape=jax.ShapeDtypeStruct((2, N, D_HID // 2), jnp.float32),
)


def _tc2_body(raw_ref, deg_ref, b1_ref, w2_ref, out_ref):
    dis = _dis_from(deg_ref)
    raw = raw_ref[...]
    h = jnp.concatenate([raw[0], raw[1]], axis=1) * dis[:, None] + b1_ref[...]
    h = jnp.maximum(h, 0.0)
    h2 = jnp.dot(h, w2_ref[...], preferred_element_type=jnp.float32)
    out_ref[...] = h2 * dis[:, None]


_tc2 = pl.pallas_call(
    _tc2_body,
    grid=(N // _RB,),
    in_specs=[
        pl.BlockSpec((2, _RB, D_HID // 2), lambda i: (0, i, 0)),
        pl.BlockSpec((_RB, 2), lambda i: (i, 0)),
        pl.BlockSpec((1, D_HID), lambda i: (0, 0)),
        pl.BlockSpec((D_HID, D_OUT), lambda i: (0, 0)),
    ],
    out_specs=pl.BlockSpec((_RB, D_OUT), lambda i: (i, 0)),
    out_shape=jax.ShapeDtypeStruct((N, D_OUT), jnp.float32),
)


def _tc3_body(raw_ref, deg_ref, b2_ref, out_ref):
    dis = _dis_from(deg_ref)
    raw = raw_ref[...]
    out_ref[...] = (raw[0] + raw[1]) * dis[:, None] + b2_ref[...]


_tc3 = pl.pallas_call(
    _tc3_body,
    grid=(N // _RB,),
    in_specs=[
        pl.BlockSpec((2, _RB, D_OUT), lambda i: (0, i, 0)),
        pl.BlockSpec((_RB, 2), lambda i: (i, 0)),
        pl.BlockSpec((1, D_OUT), lambda i: (0, 0)),
    ],
    out_specs=pl.BlockSpec((_RB, D_OUT), lambda i: (i, 0)),
    out_shape=jax.ShapeDtypeStruct((N, D_OUT), jnp.float32),
)


def kernel(x, edge_index, W1, b1, W2, b2):
    ei = edge_index.astype(jnp.int32)
    src = ei[0]
    dst = ei[1]

    h1 = _tc0(x, W1)
    deg_pad = _deg_sc(dst)
    deg2 = deg_pad.reshape(NC, ACC_ROWS)[:, :N].T
    h1s_split = _tc1(h1, deg2)
    table1 = h1s_split.reshape(2 * N, D_HID // 2)
    raw1_pad = _agg1(src, dst, table1)
    raw1 = raw1_pad.reshape(NC, ACC_ROWS, D_HID // 2)[:, :N]

    h2s = _tc2(raw1, deg2, b1.reshape(1, D_HID), W2)
    raw2_pad = _agg2(src, dst, h2s)
    raw2 = raw2_pad.reshape(NC, ACC_ROWS, D_OUT)[:, :N]

    return _tc3(raw2, deg2, b2.reshape(1, D_OUT))
